# baseline (speedup 1.0000x reference)
import functools

import jax
import jax.numpy as jnp
from jax.experimental import pallas as pl
from jax.experimental.pallas import tpu as pltpu
from jax.experimental.pallas import tpu_sc as plsc

MAXLEN = 1024


def _interval_count_mask(r, starts_ref, ends_ref, nb):
    cnt = jnp.zeros(r.shape, jnp.int32)
    for b in range(nb):
        s = starts_ref[b]
        e = ends_ref[b]
        cnt = cnt + (r >= s).astype(jnp.int32) - (r >= e).astype(jnp.int32)
    return (cnt > 0).astype(jnp.float32)


def _node_mask_body(starts_ref, ends_ref, out_ref):
    g, lanes = out_ref.shape
    i = pl.program_id(0)
    r = (i * g + jax.lax.broadcasted_iota(jnp.int32, (g, lanes), 0)) * lanes \
        + jax.lax.broadcasted_iota(jnp.int32, (g, lanes), 1)
    out_ref[...] = _interval_count_mask(r, starts_ref, ends_ref,
                                        starts_ref.shape[0])


def _atom_mask_body(starts_ref, ends_ref, a2r_ref, out_ref):
    r = a2r_ref[...]
    out_ref[...] = _interval_count_mask(r, starts_ref, ends_ref,
                                        starts_ref.shape[0])


def _build_atom_mask(starts, ends, atom2residue, A):
    LANES = 128
    GA = 256
    a2r2d = atom2residue.reshape(A // LANES, LANES)
    atom_mask = pl.pallas_call(
        _atom_mask_body,
        grid=(A // (GA * LANES),),
        in_specs=[
            pl.BlockSpec(memory_space=pltpu.SMEM),
            pl.BlockSpec(memory_space=pltpu.SMEM),
            pl.BlockSpec((GA, LANES), lambda i: (i, 0)),
        ],
        out_specs=pl.BlockSpec((GA, LANES), lambda i: (i, 0)),
        out_shape=jax.ShapeDtypeStruct((A // LANES, LANES), jnp.float32),
    )(starts, ends, a2r2d)
    return atom_mask.reshape(A, 1)


NBUF = 4


def _skip_mul_body(starts_ref, ends_ref, rmin_ref, rmax_ref,
                   mask_ref, feat_hbm, out_ref, scratch, sems):
    i = pl.program_id(0)
    n = pl.num_programs(0)
    nb = starts_ref.shape[0]
    br = out_ref.shape[0]

    def nonzero(j):
        acc = None
        for b in range(nb):
            hit = (starts_ref[b] <= rmax_ref[j]) & (ends_ref[b] > rmin_ref[j])
            acc = hit if acc is None else (acc | hit)
        return acc

    def start_dma(j, slot):
        pltpu.make_async_copy(
            feat_hbm.at[pl.ds(j * br, br), :], scratch.at[slot],
            sems.at[slot]).start()

    @pl.when(i == 0)
    def _():
        for j in range(NBUF - 1):
            @pl.when(jnp.logical_and(j < n, nonzero(jnp.minimum(j, n - 1))))
            def _():
                start_dma(j, j % NBUF)

    nxt = jnp.minimum(i + NBUF - 1, n - 1)

    @pl.when(jnp.logical_and(i + NBUF - 1 < n, nonzero(nxt)))
    def _():
        start_dma(nxt, jax.lax.rem(i + NBUF - 1, NBUF))

    live = nonzero(i)

    @pl.when(live)
    def _():
        slot = jax.lax.rem(i, NBUF)
        pltpu.make_async_copy(
            feat_hbm.at[pl.ds(i * br, br), :], scratch.at[slot],
            sems.at[slot]).wait()
        out_ref[...] = scratch[slot] * mask_ref[...]

    @pl.when(jnp.logical_not(live))
    def _():
        out_ref[...] = jnp.zeros_like(out_ref)


def _masked_mul_skip(feat, mask_col, starts, ends, rmin, rmax, block_rows):
    n, d = feat.shape
    grid = n // block_rows
    return pl.pallas_call(
        _skip_mul_body,
        grid_spec=pltpu.PrefetchScalarGridSpec(
            num_scalar_prefetch=4,
            grid=(grid,),
            in_specs=[
                pl.BlockSpec((block_rows, 1), lambda i, *_: (i, 0)),
                pl.BlockSpec(memory_space=pltpu.MemorySpace.HBM),
            ],
            out_specs=pl.BlockSpec((block_rows, d), lambda i, *_: (i, 0)),
            scratch_shapes=[
                pltpu.VMEM((NBUF, block_rows, d), feat.dtype),
                pltpu.SemaphoreType.DMA((NBUF,)),
            ],
        ),
        out_shape=jax.ShapeDtypeStruct((n, d), feat.dtype),
        compiler_params=pltpu.CompilerParams(
            dimension_semantics=("arbitrary",)),
    )(starts, ends, rmin, rmax, mask_col, feat)


def _sc_masked_copy(feat_flat, starts_pad, ends_pad, n_rows, d):
    info = plsc.get_sparse_core_info()
    nw = info.num_cores * info.num_subcores
    nb = starts_pad.shape[0] - 16
    rows_per_w = n_rows // nw
    CH = 128
    nchunks = rows_per_w // CH
    mesh = plsc.VectorSubcoreMesh(core_axis_name="c", subcore_axis_name="s")

    @functools.partial(
        pl.kernel, mesh=mesh,
        compiler_params=pltpu.CompilerParams(needs_layout_passes=False),
        out_type=jax.ShapeDtypeStruct((n_rows * d,), jnp.float32),
        scratch_types=[
            pltpu.VMEM((CH * d,), jnp.float32),
            pltpu.VMEM((16 + CH,), jnp.float32),
            pltpu.VMEM((16 + nb,), jnp.int32),
            pltpu.VMEM((16 + nb,), jnp.int32),
        ],
    )
    def k(feat_hbm, starts_hbm, ends_hbm, out_hbm, buf, mbuf, s_v, e_v):
        wid = jax.lax.axis_index("s") * info.num_cores + jax.lax.axis_index("c")
        base_row = wid * rows_per_w
        pltpu.sync_copy(starts_hbm, s_v)
        pltpu.sync_copy(ends_hbm, e_v)

        s_spl = [plsc.load_gather(s_v, [jnp.full((16,), 16 + b, jnp.int32)])
                 for b in range(nb)]
        e_spl = [plsc.load_gather(e_v, [jnp.full((16,), 16 + b, jnp.int32)])
                 for b in range(nb)]
        sh31 = jnp.full((16,), 31, jnp.int32)
        one16 = jnp.full((16,), 1, jnp.int32)

        def chunk_body(c, _):
            row0 = base_row + c * CH
            pltpu.sync_copy(feat_hbm.at[pl.ds(row0 * d, CH * d)], buf)

            def grp_body(v, _):
                r = row0 + v * 16 + jax.lax.iota(jnp.int32, 16)
                cnt = jnp.zeros((16,), jnp.int32)
                for b in range(nb):
                    cnt = cnt \
                        + jax.lax.shift_right_arithmetic(r - s_spl[b], sh31) \
                        - jax.lax.shift_right_arithmetic(r - e_spl[b], sh31)
                mbuf[pl.ds(16 + v * 16, 16)] = \
                    jnp.minimum(cnt, one16).astype(jnp.float32)

                for j in range(16):
                    row = v * 16 + j
                    mrow = plsc.load_gather(
                        mbuf, [jnp.full((16,), 16 + row, jnp.int32)])
                    off = row * d
                    for kk in range(d // 16):
                        sl = pl.ds(off + kk * 16, 16)
                        buf[sl] = buf[sl] * mrow
                return 0

            jax.lax.fori_loop(0, CH // 16, grp_body, 0)
            pltpu.sync_copy(buf, out_hbm.at[pl.ds(row0 * d, CH * d)])
            return 0

        jax.lax.fori_loop(0, nchunks, chunk_body, 0)

    return k(feat_flat, starts_pad, ends_pad)


def kernel(residue_feat, atom_feat, rand_u, num_residues, atom2residue):
    L, D = residue_feat.shape
    A = atom_feat.shape[0]
    num_cum = jnp.cumsum(num_residues)
    starts_local = (rand_u * jnp.clip(num_residues - MAXLEN, 0, None)
                    .astype(jnp.float32)).astype(jnp.int32)
    ends_local = jnp.minimum(starts_local + MAXLEN, num_residues)
    offset = num_cum - num_residues
    starts = starts_local + offset
    ends = ends_local + offset

    out_residue = _sc_masked_copy(residue_feat.reshape(L * D),
                                  jnp.pad(starts, (16, 0)),
                                  jnp.pad(ends, (16, 0)),
                                  L, D).reshape(L, D)

    atom_mask = _build_atom_mask(starts, ends, atom2residue, A)
    BA = 2048
    atom_rmin = atom2residue[0::BA]
    atom_rmax = atom2residue[BA - 1::BA]
    out_atom = _masked_mul_skip(atom_feat, atom_mask, starts, ends,
                                atom_rmin, atom_rmax, BA)
    return out_residue, out_atom


# speedup vs baseline: 1.0013x; 1.0013x over previous
import functools

import jax
import jax.numpy as jnp
from jax.experimental import pallas as pl
from jax.experimental.pallas import tpu as pltpu
from jax.experimental.pallas import tpu_sc as plsc

MAXLEN = 1024


def _interval_count_mask(r, starts_ref, ends_ref, nb):
    cnt = jnp.zeros(r.shape, jnp.int32)
    for b in range(nb):
        s = starts_ref[b]
        e = ends_ref[b]
        cnt = cnt + (r >= s).astype(jnp.int32) - (r >= e).astype(jnp.int32)
    return (cnt > 0).astype(jnp.float32)


def _node_mask_body(starts_ref, ends_ref, out_ref):
    g, lanes = out_ref.shape
    i = pl.program_id(0)
    r = (i * g + jax.lax.broadcasted_iota(jnp.int32, (g, lanes), 0)) * lanes \
        + jax.lax.broadcasted_iota(jnp.int32, (g, lanes), 1)
    out_ref[...] = _interval_count_mask(r, starts_ref, ends_ref,
                                        starts_ref.shape[0])


def _atom_mask_body(starts_ref, ends_ref, a2r_ref, out_ref):
    r = a2r_ref[...]
    out_ref[...] = _interval_count_mask(r, starts_ref, ends_ref,
                                        starts_ref.shape[0])


def _build_atom_mask(starts, ends, atom2residue, A):
    LANES = 128
    GA = 256
    a2r2d = atom2residue.reshape(A // LANES, LANES)
    atom_mask = pl.pallas_call(
        _atom_mask_body,
        grid=(A // (GA * LANES),),
        in_specs=[
            pl.BlockSpec(memory_space=pltpu.SMEM),
            pl.BlockSpec(memory_space=pltpu.SMEM),
            pl.BlockSpec((GA, LANES), lambda i: (i, 0)),
        ],
        out_specs=pl.BlockSpec((GA, LANES), lambda i: (i, 0)),
        out_shape=jax.ShapeDtypeStruct((A // LANES, LANES), jnp.float32),
    )(starts, ends, a2r2d)
    return atom_mask.reshape(A, 1)


NBUF = 4


def _skip_mul_body(starts_ref, ends_ref, rmin_ref, rmax_ref,
                   mask_ref, feat_hbm, out_ref, scratch, sems):
    i = pl.program_id(0)
    n = pl.num_programs(0)
    nb = starts_ref.shape[0]
    br = out_ref.shape[0]

    def nonzero(j):
        acc = None
        for b in range(nb):
            hit = (starts_ref[b] <= rmax_ref[j]) & (ends_ref[b] > rmin_ref[j])
            acc = hit if acc is None else (acc | hit)
        return acc

    def start_dma(j, slot):
        pltpu.make_async_copy(
            feat_hbm.at[pl.ds(j * br, br), :], scratch.at[slot],
            sems.at[slot]).start()

    @pl.when(i == 0)
    def _():
        for j in range(NBUF - 1):
            @pl.when(jnp.logical_and(j < n, nonzero(jnp.minimum(j, n - 1))))
            def _():
                start_dma(j, j % NBUF)

    nxt = jnp.minimum(i + NBUF - 1, n - 1)

    @pl.when(jnp.logical_and(i + NBUF - 1 < n, nonzero(nxt)))
    def _():
        start_dma(nxt, jax.lax.rem(i + NBUF - 1, NBUF))

    live = nonzero(i)

    @pl.when(live)
    def _():
        slot = jax.lax.rem(i, NBUF)
        pltpu.make_async_copy(
            feat_hbm.at[pl.ds(i * br, br), :], scratch.at[slot],
            sems.at[slot]).wait()
        out_ref[...] = scratch[slot] * mask_ref[...]

    @pl.when(jnp.logical_not(live))
    def _():
        out_ref[...] = jnp.zeros_like(out_ref)


def _masked_mul_skip(feat, mask_col, starts, ends, rmin, rmax, block_rows):
    n, d = feat.shape
    grid = n // block_rows
    return pl.pallas_call(
        _skip_mul_body,
        grid_spec=pltpu.PrefetchScalarGridSpec(
            num_scalar_prefetch=4,
            grid=(grid,),
            in_specs=[
                pl.BlockSpec((block_rows, 1), lambda i, *_: (i, 0)),
                pl.BlockSpec(memory_space=pltpu.MemorySpace.HBM),
            ],
            out_specs=pl.BlockSpec((block_rows, d), lambda i, *_: (i, 0)),
            scratch_shapes=[
                pltpu.VMEM((NBUF, block_rows, d), feat.dtype),
                pltpu.SemaphoreType.DMA((NBUF,)),
            ],
        ),
        out_shape=jax.ShapeDtypeStruct((n, d), feat.dtype),
        compiler_params=pltpu.CompilerParams(
            dimension_semantics=("arbitrary",)),
    )(starts, ends, rmin, rmax, mask_col, feat)


def _sc_masked_copy(feat_flat, starts_pad, ends_pad, n_rows, d):
    info = plsc.get_sparse_core_info()
    nw = info.num_cores * info.num_subcores
    nb = starts_pad.shape[0] - 16
    rows_per_w = n_rows // nw
    CH = 128
    nchunks = rows_per_w // CH
    mesh = plsc.VectorSubcoreMesh(core_axis_name="c", subcore_axis_name="s")

    @functools.partial(
        pl.kernel, mesh=mesh,
        compiler_params=pltpu.CompilerParams(needs_layout_passes=False),
        out_type=jax.ShapeDtypeStruct((n_rows * d,), jnp.float32),
        scratch_types=[
            pltpu.VMEM((CH * d,), jnp.float32),
            pltpu.VMEM((16 + CH,), jnp.float32),
            pltpu.VMEM((16 + nb,), jnp.int32),
            pltpu.VMEM((16 + nb,), jnp.int32),
        ],
    )
    def k(feat_hbm, starts_hbm, ends_hbm, out_hbm, buf, mbuf, s_v, e_v):
        wid = jax.lax.axis_index("s") * info.num_cores + jax.lax.axis_index("c")
        base_row = wid * rows_per_w
        pltpu.sync_copy(starts_hbm, s_v)
        pltpu.sync_copy(ends_hbm, e_v)

        s_spl = [plsc.load_gather(s_v, [jnp.full((16,), 16 + b, jnp.int32)])
                 for b in range(nb)]
        e_spl = [plsc.load_gather(e_v, [jnp.full((16,), 16 + b, jnp.int32)])
                 for b in range(nb)]
        sh31 = jnp.full((16,), 31, jnp.int32)
        one16 = jnp.full((16,), 1, jnp.int32)

        def chunk_body(c, _):
            row0 = base_row + c * CH
            pltpu.sync_copy(feat_hbm.at[pl.ds(row0 * d, CH * d)], buf)

            def grp_body(v, _):
                r = row0 + v * 16 + jax.lax.iota(jnp.int32, 16)
                cnt = jnp.zeros((16,), jnp.int32)
                for b in range(nb):
                    cnt = cnt \
                        + jax.lax.shift_right_arithmetic(r - s_spl[b], sh31) \
                        - jax.lax.shift_right_arithmetic(r - e_spl[b], sh31)
                mbuf[pl.ds(16 + v * 16, 16)] = \
                    jnp.minimum(cnt, one16).astype(jnp.float32)

                for j in range(16):
                    row = v * 16 + j
                    mrow = plsc.load_gather(
                        mbuf, [jnp.full((16,), 16 + row, jnp.int32)])
                    off = row * d
                    for kk in range(d // 16):
                        sl = pl.ds(off + kk * 16, 16)
                        buf[sl] = buf[sl] * mrow
                return 0

            jax.lax.fori_loop(0, CH // 16, grp_body, 0)
            pltpu.sync_copy(buf, out_hbm.at[pl.ds(row0 * d, CH * d)])
            return 0

        jax.lax.fori_loop(0, nchunks, chunk_body, 0)

    return k(feat_flat, starts_pad, ends_pad)


def kernel(residue_feat, atom_feat, rand_u, num_residues, atom2residue):
    L, D = residue_feat.shape
    A = atom_feat.shape[0]
    num_cum = jnp.cumsum(num_residues)
    starts_local = (rand_u * jnp.clip(num_residues - MAXLEN, 0, None)
                    .astype(jnp.float32)).astype(jnp.int32)
    ends_local = jnp.minimum(starts_local + MAXLEN, num_residues)
    offset = num_cum - num_residues
    starts = starts_local + offset
    ends = ends_local + offset

    atom_mask = _build_atom_mask(starts, ends, atom2residue, A)

    BA = 2048
    atom_rmin = atom2residue[0::BA]
    atom_rmax = atom2residue[BA - 1::BA]

    out_residue = _sc_masked_copy(residue_feat.reshape(L * D),
                                  jnp.pad(starts, (16, 0)),
                                  jnp.pad(ends, (16, 0)),
                                  L, D).reshape(L, D)
    out_atom = _masked_mul_skip(atom_feat, atom_mask, starts, ends,
                                atom_rmin, atom_rmax, BA)
    return out_residue, out_atom
